# baseline (speedup 1.0000x reference)
import functools

import jax
import jax.numpy as jnp
from jax import lax
from jax.experimental import pallas as pl
from jax.experimental.pallas import tpu as pltpu
from jax.experimental.pallas import tpu_sc as plsc

_NUM_CLASSES = 1000
_HIDDEN = 128
_BATCH = 16384
_ROWS = _NUM_CLASSES + 1

_INFO = plsc.get_sparse_core_info()
_NC = _INFO.num_cores
_NS = _INFO.num_subcores
_L = _INFO.num_lanes
_NW = _NC * _NS
_B_PER_W = _BATCH // _NW
_CHUNK = 64
_NCHUNK = _B_PER_W // _CHUNK
_VPC = _CHUNK // _L
_STAGE = 64
_TAIL_ROWS = _ROWS - (_NS - 1) * _STAGE
_NPAD = _NS * _STAGE
_EXT_ROWS = _NPAD + _NS * _STAGE
_HBM_CHUNKS = (5, 6, 7)

_mesh = plsc.VectorSubcoreMesh(core_axis_name="c", subcore_axis_name="s")


@functools.partial(
    pl.kernel,
    mesh=_mesh,
    out_type=(
        jax.ShapeDtypeStruct((_BATCH, _HIDDEN), jnp.float32),
        jax.ShapeDtypeStruct((_NC, _EXT_ROWS, _HIDDEN), jnp.float32),
    ),
    scratch_types=[
        pltpu.VMEM_SHARED((_NS * _STAGE, _HIDDEN), jnp.float32),
        pltpu.VMEM((_B_PER_W,), jnp.int32),
        pltpu.VMEM((_B_PER_W,), jnp.int32),
        pltpu.VMEM((_B_PER_W,), jnp.int32),
        pltpu.VMEM((_B_PER_W, _HIDDEN), jnp.float32),
        pltpu.VMEM((_STAGE, _HIDDEN), jnp.float32),
        pltpu.SemaphoreType.DMA,
        pltpu.SemaphoreType.DMA((_NCHUNK,)),
        pltpu.SemaphoreType.DMA,
        pltpu.SemaphoreType.DMA,
    ],
)
def _embed(
    labels_hbm,
    drop_hbm,
    table_hbm,
    out_hbm,
    ext_hbm,
    table_sp,
    lab_v,
    drop_v,
    idx_v,
    rows_v,
    rep_v,
    sem_t,
    sem_g,
    sem_w,
    sem_e,
):
    cid = lax.axis_index("c")
    sid = lax.axis_index("s")
    wid = sid * _NC + cid
    base = wid * _B_PER_W
    start = pl.multiple_of(sid * _STAGE, 8)
    tail = (_NS - 1) * _STAGE

    @pl.when(sid < _NS - 1)
    def _stage_body():
        pltpu.async_copy(
            table_hbm.at[pl.ds(start, _STAGE)], table_sp.at[pl.ds(start, _STAGE)], sem_t
        )

    @pl.when(sid == _NS - 1)
    def _stage_tail():
        pltpu.async_copy(
            table_hbm.at[pl.ds(tail, _TAIL_ROWS)],
            table_sp.at[pl.ds(tail, _TAIL_ROWS)],
            sem_t,
        )

    lab_copy = pltpu.async_copy(labels_hbm.at[pl.ds(base, _B_PER_W)], lab_v, sem_w)
    drop_copy = pltpu.async_copy(drop_hbm.at[pl.ds(base, _B_PER_W)], drop_v, sem_w)
    lab_copy.wait()
    drop_copy.wait()

    def _adjust_sp(i, carry):
        sl0 = pl.ds(pl.multiple_of(i * 2 * _L, _L), _L)
        sl1 = pl.ds(pl.multiple_of(i * 2 * _L + _L, _L), _L)
        idx_v[sl0] = jnp.where(drop_v[sl0] == 1, _NUM_CLASSES, lab_v[sl0])
        idx_v[sl1] = jnp.where(drop_v[sl1] == 1, _NUM_CLASSES, lab_v[sl1])
        return carry

    lax.fori_loop(0, _HBM_CHUNKS[0] * _VPC // 2, _adjust_sp, 0)

    lane = lax.iota(jnp.int32, _L)

    def _adjust_hbm(i, carry):
        pos = _HBM_CHUNKS[0] * _CHUNK + i * _L
        sl = pl.ds(pl.multiple_of(pos, _L), _L)
        spread = (base + pos + lane) & (_NPAD - 1)
        idx_v[sl] = jnp.where(drop_v[sl] == 1, spread, _NPAD + lab_v[sl])
        return carry

    lax.fori_loop(0, (_NCHUNK - _HBM_CHUNKS[0]) * _VPC, _adjust_hbm, 0)

    @pl.when(sid < _NS - 1)
    def _wait_body():
        pltpu.make_async_copy(
            table_hbm.at[pl.ds(start, _STAGE)], table_sp.at[pl.ds(start, _STAGE)], sem_t
        ).wait()

    @pl.when(sid == _NS - 1)
    def _wait_tail():
        pltpu.make_async_copy(
            table_hbm.at[pl.ds(tail, _TAIL_ROWS)],
            table_sp.at[pl.ds(tail, _TAIL_ROWS)],
            sem_t,
        ).wait()

    plsc.subcore_barrier()

    gathers = [None] * _NCHUNK
    for j in range(_NCHUNK):
        if j in _HBM_CHUNKS:
            continue
        gathers[j] = pltpu.async_copy(
            table_sp.at[idx_v.at[pl.ds(j * _CHUNK, _CHUNK)]],
            rows_v.at[pl.ds(j * _CHUNK, _CHUNK)],
            sem_g.at[j],
        )

    ext_core = ext_hbm.at[cid]
    real_copy = pltpu.async_copy(
        table_sp.at[pl.ds(start, _STAGE)],
        ext_core.at[pl.ds(_NPAD + start, _STAGE)],
        sem_e,
    )
    pltpu.sync_copy(table_sp.at[pl.ds(_NUM_CLASSES, 8)], rep_v.at[pl.ds(0, 8)])
    cfg_regs = [rep_v[0, pl.ds(k * _L, _L)] for k in range(_HIDDEN // _L)]

    def _replicate(r, carry):
        for k in range(_HIDDEN // _L):
            rep_v[r, pl.ds(k * _L, _L)] = cfg_regs[k]
        return carry

    lax.fori_loop(1, _STAGE, _replicate, 0)
    rep_copy = pltpu.async_copy(rep_v, ext_core.at[pl.ds(start, _STAGE)], sem_e)
    real_copy.wait()
    rep_copy.wait()
    plsc.subcore_barrier()

    for j in _HBM_CHUNKS:
        gathers[j] = pltpu.async_copy(
            ext_core.at[idx_v.at[pl.ds(j * _CHUNK, _CHUNK)]],
            rows_v.at[pl.ds(j * _CHUNK, _CHUNK)],
            sem_g.at[j],
        )

    writebacks = []
    for j in range(_NCHUNK):
        gathers[j].wait()
        writebacks.append(
            pltpu.async_copy(
                rows_v.at[pl.ds(j * _CHUNK, _CHUNK)],
                out_hbm.at[pl.ds(base + j * _CHUNK, _CHUNK)],
                sem_w,
            )
        )
    for wb in writebacks:
        wb.wait()


def kernel(labels, train, force_drop_ids, embedding_table):
    del train
    out, _ = _embed(labels, force_drop_ids, embedding_table)
    return out


# speedup vs baseline: 1.1563x; 1.1563x over previous
import functools

import jax
import jax.numpy as jnp
from jax import lax
from jax.experimental import pallas as pl
from jax.experimental.pallas import tpu as pltpu
from jax.experimental.pallas import tpu_sc as plsc

_NUM_CLASSES = 1000
_HIDDEN = 128
_BATCH = 16384
_ROWS = _NUM_CLASSES + 1

_INFO = plsc.get_sparse_core_info()
_NC = _INFO.num_cores
_NS = _INFO.num_subcores
_L = _INFO.num_lanes
_NW = _NC * _NS
_B_PER_W = _BATCH // _NW
_CHUNK = 64
_NCHUNK = _B_PER_W // _CHUNK
_STAGE = 64

_mesh = plsc.VectorSubcoreMesh(core_axis_name="c", subcore_axis_name="s")


@functools.partial(
    pl.kernel,
    mesh=_mesh,
    out_type=jax.ShapeDtypeStruct((_BATCH, _HIDDEN), jnp.float32),
    scratch_types=[
        pltpu.VMEM_SHARED((_ROWS, _HIDDEN), jnp.float32),
        pltpu.VMEM((_B_PER_W,), jnp.int32),
        pltpu.VMEM((_B_PER_W,), jnp.int32),
        pltpu.VMEM((_B_PER_W,), jnp.int32),
        pltpu.VMEM((_B_PER_W, _HIDDEN), jnp.float32),
        pltpu.SemaphoreType.DMA,
        pltpu.SemaphoreType.DMA((_NCHUNK,)),
        pltpu.SemaphoreType.DMA,
    ],
)
def _embed(
    labels_hbm,
    drop_hbm,
    table_hbm,
    out_hbm,
    table_sp,
    lab_v,
    drop_v,
    idx_v,
    rows_v,
    sem_t,
    sem_g,
    sem_w,
):
    sid = lax.axis_index("s")
    wid = sid * _NC + lax.axis_index("c")
    base = wid * _B_PER_W
    @pl.when(sid < _NS - 1)
    def _stage_body():
        start = pl.multiple_of(sid * _STAGE, 8)
        pltpu.async_copy(
            table_hbm.at[pl.ds(start, _STAGE)], table_sp.at[pl.ds(start, _STAGE)], sem_t
        )

    tail = (_NS - 1) * _STAGE

    @pl.when(sid == _NS - 1)
    def _stage_tail():
        pltpu.async_copy(
            table_hbm.at[pl.ds(tail, _ROWS - tail)],
            table_sp.at[pl.ds(tail, _ROWS - tail)],
            sem_t,
        )

    pltpu.sync_copy(labels_hbm.at[pl.ds(base, _B_PER_W)], lab_v)
    pltpu.sync_copy(drop_hbm.at[pl.ds(base, _B_PER_W)], drop_v)

    def _adjust(i, carry):
        sl = pl.ds(pl.multiple_of(i * _L, _L), _L)
        idx_v[sl] = jnp.where(drop_v[sl] == 1, _NUM_CLASSES, lab_v[sl])
        return carry

    lax.fori_loop(0, _B_PER_W // _L, _adjust, 0)

    @pl.when(sid < _NS - 1)
    def _wait_body():
        start = pl.multiple_of(sid * _STAGE, 8)
        pltpu.make_async_copy(
            table_hbm.at[pl.ds(start, _STAGE)], table_sp.at[pl.ds(start, _STAGE)], sem_t
        ).wait()

    @pl.when(sid == _NS - 1)
    def _wait_tail():
        pltpu.make_async_copy(
            table_hbm.at[pl.ds(tail, _ROWS - tail)],
            table_sp.at[pl.ds(tail, _ROWS - tail)],
            sem_t,
        ).wait()

    plsc.subcore_barrier()

    gathers = []
    for j in range(_NCHUNK):
        gathers.append(
            pltpu.async_copy(
                table_sp.at[idx_v.at[pl.ds(j * _CHUNK, _CHUNK)]],
                rows_v.at[pl.ds(j * _CHUNK, _CHUNK)],
                sem_g.at[j],
            )
        )
    writebacks = []
    for j in range(_NCHUNK):
        gathers[j].wait()
        writebacks.append(
            pltpu.async_copy(
                rows_v.at[pl.ds(j * _CHUNK, _CHUNK)],
                out_hbm.at[pl.ds(base + j * _CHUNK, _CHUNK)],
                sem_w,
            )
        )
    for wb in writebacks:
        wb.wait()


def kernel(labels, train, force_drop_ids, embedding_table):
    del train
    return _embed(labels, force_drop_ids, embedding_table)
